# baseline (speedup 1.0000x reference)
import functools

import jax
import jax.numpy as jnp
from jax import lax
from jax.experimental import pallas as pl
from jax.experimental.pallas import tpu as pltpu
from jax.experimental.pallas import tpu_sc as plsc

KNN = 12
NC, NS, L = 2, 16, 16
NW = NC * NS


def _knn_body(q_ref, xT_ref, o_idx_ref, o_w_ref):
    b = pl.program_id(0)
    q = q_ref[0]
    xT = xT_ref[0]
    TM = q.shape[0]
    N = xT.shape[1]

    qx = jnp.dot(q, xT, preferred_element_type=jnp.float32)
    q2 = jnp.sum(q * q, axis=1, keepdims=True)
    x2 = jnp.sum(xT * xT, axis=0, keepdims=True)
    d2 = jnp.maximum(q2 + x2 - 2.0 * qx, 0.0)

    SIGN = jnp.int32(-2**31)
    colx = lax.broadcasted_iota(jnp.int32, (TM, N), 1) | SIGN
    key = (lax.bitcast_convert_type(d2, jnp.int32) & jnp.int32(~(N - 1))
           ) | colx

    off = jnp.zeros((TM, 1), jnp.int32)
    us = []
    for _ in range(KNN):
        m = jnp.min(key - off, axis=1, keepdims=True)
        off = off + m + jnp.int32(-2**31 + 1)
        us.append(off - 1)
    u = jnp.concatenate(us, axis=1)

    idx = (u & jnp.int32(N - 1)) + b * N
    d2q = lax.bitcast_convert_type(u & jnp.int32(~(N - 1)), jnp.float32)
    w = 1.0 / (d2q + 1e-8)
    wn = w * (1.0 / jnp.sum(w, axis=1, keepdims=True))

    o_idx_ref[0] = jnp.concatenate([idx] + [idx[:, :1]] * 4, axis=1)
    o_w_ref[0] = jnp.concatenate(
        [wn, jnp.zeros((TM, 4), jnp.float32)], axis=1)


def _knn_call(qpad, xT, B, M, N, TM):
    return pl.pallas_call(
        _knn_body,
        grid=(B, M // TM),
        in_specs=[
            pl.BlockSpec((1, TM, 8), lambda b, m: (b, m, 0)),
            pl.BlockSpec((1, 8, N), lambda b, m: (b, 0, 0)),
        ],
        out_specs=[
            pl.BlockSpec((1, TM, 16), lambda b, m: (b, m, 0)),
            pl.BlockSpec((1, TM, 16), lambda b, m: (b, m, 0)),
        ],
        out_shape=[
            jax.ShapeDtypeStruct((B, M, 16), jnp.int32),
            jax.ShapeDtypeStruct((B, M, 16), jnp.float32),
        ],
        compiler_params=pltpu.CompilerParams(
            dimension_semantics=("parallel", "parallel")),
    )(qpad, xT)


def _make_sc_gather(BM, C, rows_total):
    rows_per_w = rows_total // NW
    mesh = plsc.VectorSubcoreMesh(core_axis_name="c", subcore_axis_name="s")

    @functools.partial(
        pl.kernel, mesh=mesh,
        out_type=jax.ShapeDtypeStruct((BM, C), jnp.float32),
        scratch_types=[
            pltpu.VMEM((128,), jnp.int32),
            pltpu.VMEM((128,), jnp.float32),
            pltpu.VMEM((128, C), jnp.float32),
            pltpu.VMEM((8, C), jnp.float32),
            pltpu.SemaphoreType.DMA,
        ],
    )
    def sc_gather(f5_hbm, idx_hbm, w_hbm, out_hbm, idx_v, w_v, rows_v,
                  out_v, sem):
        wid = lax.axis_index("s") * NC + lax.axis_index("c")
        base = wid * rows_per_w

        def row_body(j, carry):
            r = base + j
            pltpu.sync_copy(idx_hbm.at[r], idx_v)
            pltpu.sync_copy(w_hbm.at[r], w_v)
            pltpu.async_copy(f5_hbm.at[idx_v], rows_v, sem).wait()

            def q_body(q, carry2):
                qb = q * 16
                wvec = w_v[pl.ds(qb, 16)]
                acc = [jnp.zeros((L,), jnp.float32) for _ in range(C // L)]
                for rr in range(KNN):
                    wr = wvec[rr]
                    for c in range(C // L):
                        acc[c] = acc[c] + wr * rows_v[qb + rr,
                                                      pl.ds(c * L, L)]
                for c in range(C // L):
                    out_v[q, pl.ds(c * L, L)] = acc[c]
                return carry2

            lax.fori_loop(0, 8, q_body, 0)
            pltpu.sync_copy(out_v, out_hbm.at[pl.ds(r * 8, 8)])
            return carry

        lax.fori_loop(0, rows_per_w, row_body, 0)

    return sc_gather


def _mlp_body(x_ref, Wi_ref, bi_ref, W1_ref, b1_ref, W2_ref, b2_ref, o_ref):
    h0 = jnp.maximum(jnp.dot(x_ref[...].astype(jnp.bfloat16), Wi_ref[...],
                             preferred_element_type=jnp.float32)
                     + bi_ref[...], 0.0)
    h1 = jnp.maximum(jnp.dot(h0.astype(jnp.bfloat16), W1_ref[...],
                             preferred_element_type=jnp.float32)
                     + b1_ref[...], 0.0)
    o_ref[...] = jnp.tanh(jnp.dot(h1, W2_ref[...],
                                  preferred_element_type=jnp.float32)
                          + b2_ref[...])


def _mlp_call(x, Wi, bi2, W1, b12, W2p, b2p, BM, C, H, TX=512):
    return pl.pallas_call(
        _mlp_body,
        grid=(BM // TX,),
        in_specs=[
            pl.BlockSpec((TX, C), lambda i: (i, 0)),
            pl.BlockSpec((C, C), lambda i: (0, 0)),
            pl.BlockSpec((1, C), lambda i: (0, 0)),
            pl.BlockSpec((C, H), lambda i: (0, 0)),
            pl.BlockSpec((1, H), lambda i: (0, 0)),
            pl.BlockSpec((H, 8), lambda i: (0, 0)),
            pl.BlockSpec((1, 8), lambda i: (0, 0)),
        ],
        out_specs=pl.BlockSpec((TX, 8), lambda i: (i, 0)),
        out_shape=jax.ShapeDtypeStruct((BM, 8), jnp.float32),
        compiler_params=pltpu.CompilerParams(
            dimension_semantics=("parallel",)),
    )(x, Wi, bi2, W1, b12, W2p, b2p)


def kernel(feature4, feature5, feature6, xyz, detect_point, W_ind, b_ind,
           W1, b1, W2, b2):
    B, M, _ = detect_point.shape
    _, N, C = feature5.shape
    H = W1.shape[1]
    TM = min(256, M)
    BM = B * M

    qpad = jnp.pad(detect_point, ((0, 0), (0, 0), (0, 5)))
    xT = jnp.pad(xyz, ((0, 0), (0, 0), (0, 5))).transpose(0, 2, 1)
    W2p = jnp.pad(W2, ((0, 0), (0, 5)))
    b2p = jnp.pad(b2, ((0, 5),)).reshape(1, 8)
    bi2 = b_ind.reshape(1, -1)
    b12 = b1.reshape(1, -1)
    Wih = W_ind.astype(jnp.bfloat16)
    W1h = W1.astype(jnp.bfloat16)

    idx, w = _knn_call(qpad, xT, B, M, N, TM)
    rows_total = BM * 16 // 128
    idx2 = idx.reshape(rows_total, 128)
    w2 = w.reshape(rows_total, 128)
    f5flat = feature5.reshape(B * N, C)

    interp = _make_sc_gather(BM, C, rows_total)(f5flat, idx2, w2)

    out = _mlp_call(interp, Wih, bi2, W1h, b12, W2p, b2p, BM, C, H)
    return out.reshape(B, M, 8)[:, :, :3]


# speedup vs baseline: 1.3401x; 1.3401x over previous
import functools

import jax
import jax.numpy as jnp
from jax import lax
from jax.experimental import pallas as pl
from jax.experimental.pallas import tpu as pltpu
from jax.experimental.pallas import tpu_sc as plsc

KNN = 12
NC, NS, L = 2, 16, 16
NW = NC * NS


def _knn_body(q_ref, xT_ref, o_idx_ref, o_w_ref):
    b = pl.program_id(0)
    q = q_ref[0]
    xT = xT_ref[0]
    TM = q.shape[0]
    N = xT.shape[1]

    qx = jnp.dot(q, xT, preferred_element_type=jnp.float32)
    q2 = jnp.sum(q * q, axis=1, keepdims=True)
    x2 = jnp.sum(xT * xT, axis=0, keepdims=True)
    d2 = jnp.maximum(q2 + x2 - 2.0 * qx, 0.0)

    SIGN = jnp.int32(-2**31)
    colx = lax.broadcasted_iota(jnp.int32, (TM, N), 1) | SIGN
    key = (lax.bitcast_convert_type(d2, jnp.int32) & jnp.int32(~(N - 1))
           ) | colx

    off = jnp.zeros((TM, 1), jnp.int32)
    us = []
    for _ in range(KNN):
        m = jnp.min(key - off, axis=1, keepdims=True)
        off = off + m + jnp.int32(-2**31 + 1)
        us.append(off - 1)
    u = jnp.concatenate(us, axis=1)

    idx = (u & jnp.int32(N - 1)) + b * N
    d2q = lax.bitcast_convert_type(u & jnp.int32(~(N - 1)), jnp.float32)
    w = 1.0 / (d2q + 1e-8)
    wn = w * (1.0 / jnp.sum(w, axis=1, keepdims=True))

    o_idx_ref[0] = jnp.concatenate([idx] + [idx[:, :1]] * 4, axis=1)
    o_w_ref[0] = jnp.concatenate(
        [wn, jnp.zeros((TM, 4), jnp.float32)], axis=1)


def _knn_call(qpad, xT, B, M, N, TM):
    return pl.pallas_call(
        _knn_body,
        grid=(B, M // TM),
        in_specs=[
            pl.BlockSpec((1, TM, 8), lambda b, m: (b, m, 0)),
            pl.BlockSpec((1, 8, N), lambda b, m: (b, 0, 0)),
        ],
        out_specs=[
            pl.BlockSpec((1, TM, 16), lambda b, m: (b, m, 0)),
            pl.BlockSpec((1, TM, 16), lambda b, m: (b, m, 0)),
        ],
        out_shape=[
            jax.ShapeDtypeStruct((B, M, 16), jnp.int32),
            jax.ShapeDtypeStruct((B, M, 16), jnp.float32),
        ],
        compiler_params=pltpu.CompilerParams(
            dimension_semantics=("parallel", "parallel")),
    )(qpad, xT)


def _make_sc_gather(BM, C, chunks_total, CH):
    QC = CH // 16
    chunks_per_w = chunks_total // NW
    half = chunks_per_w // 2
    mesh = plsc.VectorSubcoreMesh(core_axis_name="c", subcore_axis_name="s")

    @functools.partial(
        pl.kernel, mesh=mesh,
        out_type=jax.ShapeDtypeStruct((BM, C), jnp.float32),
        scratch_types=[
            pltpu.VMEM((chunks_per_w, CH), jnp.int32),
            pltpu.VMEM((chunks_per_w, CH), jnp.float32),
            pltpu.VMEM((2, CH, C), jnp.float32),
            pltpu.VMEM((2, QC, C), jnp.float32),
            pltpu.SemaphoreType.DMA,
            pltpu.SemaphoreType.DMA,
            pltpu.SemaphoreType.DMA,
            pltpu.SemaphoreType.DMA,
        ],
    )
    def sc_gather(f5_hbm, idx_hbm, w_hbm, out_hbm, idx_slab, w_slab,
                  rows_v, out_v, g0, g1, s0, s1):
        wid = lax.axis_index("s") * NC + lax.axis_index("c")
        base = wid * chunks_per_w
        pltpu.sync_copy(idx_hbm.at[pl.ds(base, chunks_per_w)], idx_slab)
        pltpu.sync_copy(w_hbm.at[pl.ds(base, chunks_per_w)], w_slab)
        pltpu.async_copy(f5_hbm.at[idx_slab.at[0]], rows_v.at[0], g0)

        def compute(p, cidx, gsem, ssem):
            pltpu.make_async_copy(f5_hbm.at[idx_slab.at[cidx]],
                                  rows_v.at[p], gsem).wait()

            def q_body(q, carry):
                qb = q * 16
                wvec = w_slab[cidx, pl.ds(qb, 16)]
                acc = [jnp.zeros((L,), jnp.float32) for _ in range(C // L)]
                for rr in range(KNN):
                    wr = wvec[rr]
                    for c in range(C // L):
                        acc[c] = acc[c] + wr * rows_v[p, qb + rr,
                                                      pl.ds(c * L, L)]
                for c in range(C // L):
                    out_v[p, q, pl.ds(c * L, L)] = acc[c]
                return carry

            lax.fori_loop(0, QC, q_body, 0)
            pltpu.async_copy(out_v.at[p],
                             out_hbm.at[pl.ds((base + cidx) * QC, QC)], ssem)

        def body(k, carry):
            c0 = 2 * k
            c1 = 2 * k + 1
            pltpu.async_copy(f5_hbm.at[idx_slab.at[c1]], rows_v.at[1], g1)

            @pl.when(k > 0)
            def _drain_s0():
                pltpu.make_async_copy(out_v.at[0], out_hbm.at[pl.ds(0, QC)],
                                      s0).wait()

            compute(0, c0, g0, s0)

            @pl.when(k < half - 1)
            def _next_g0():
                pltpu.async_copy(f5_hbm.at[idx_slab.at[c0 + 2]],
                                 rows_v.at[0], g0)

            @pl.when(k > 0)
            def _drain_s1():
                pltpu.make_async_copy(out_v.at[1], out_hbm.at[pl.ds(0, QC)],
                                      s1).wait()

            compute(1, c1, g1, s1)
            return carry

        lax.fori_loop(0, half, body, 0)
        pltpu.make_async_copy(out_v.at[0], out_hbm.at[pl.ds(0, QC)],
                              s0).wait()
        pltpu.make_async_copy(out_v.at[1], out_hbm.at[pl.ds(0, QC)],
                              s1).wait()

    return sc_gather


def _mlp_body(x_ref, Wi_ref, bi_ref, W1_ref, b1_ref, W2_ref, b2_ref, o_ref):
    h0 = jnp.maximum(jnp.dot(x_ref[...].astype(jnp.bfloat16), Wi_ref[...],
                             preferred_element_type=jnp.float32)
                     + bi_ref[...], 0.0)
    h1 = jnp.maximum(jnp.dot(h0.astype(jnp.bfloat16), W1_ref[...],
                             preferred_element_type=jnp.float32)
                     + b1_ref[...], 0.0)
    o_ref[...] = jnp.tanh(jnp.dot(h1, W2_ref[...],
                                  preferred_element_type=jnp.float32)
                          + b2_ref[...])


def _mlp_call(x, Wi, bi2, W1, b12, W2p, b2p, BM, C, H, TX=512):
    return pl.pallas_call(
        _mlp_body,
        grid=(BM // TX,),
        in_specs=[
            pl.BlockSpec((TX, C), lambda i: (i, 0)),
            pl.BlockSpec((C, C), lambda i: (0, 0)),
            pl.BlockSpec((1, C), lambda i: (0, 0)),
            pl.BlockSpec((C, H), lambda i: (0, 0)),
            pl.BlockSpec((1, H), lambda i: (0, 0)),
            pl.BlockSpec((H, 8), lambda i: (0, 0)),
            pl.BlockSpec((1, 8), lambda i: (0, 0)),
        ],
        out_specs=pl.BlockSpec((TX, 8), lambda i: (i, 0)),
        out_shape=jax.ShapeDtypeStruct((BM, 8), jnp.float32),
        compiler_params=pltpu.CompilerParams(
            dimension_semantics=("parallel",)),
    )(x, Wi, bi2, W1, b12, W2p, b2p)


def kernel(feature4, feature5, feature6, xyz, detect_point, W_ind, b_ind,
           W1, b1, W2, b2):
    B, M, _ = detect_point.shape
    _, N, C = feature5.shape
    H = W1.shape[1]
    TM = min(256, M)
    BM = B * M

    qpad = jnp.pad(detect_point, ((0, 0), (0, 0), (0, 5)))
    xT = jnp.pad(xyz, ((0, 0), (0, 0), (0, 5))).transpose(0, 2, 1)
    W2p = jnp.pad(W2, ((0, 0), (0, 5)))
    b2p = jnp.pad(b2, ((0, 5),)).reshape(1, 8)
    bi2 = b_ind.reshape(1, -1)
    b12 = b1.reshape(1, -1)
    Wih = W_ind.astype(jnp.bfloat16)
    W1h = W1.astype(jnp.bfloat16)

    idx, w = _knn_call(qpad, xT, B, M, N, TM)
    CH = 64
    chunks_total = BM * 16 // CH
    idx2 = idx.reshape(chunks_total, CH)
    w2 = w.reshape(chunks_total, CH)
    f5flat = feature5.reshape(B * N, C)

    interp = _make_sc_gather(BM, C, chunks_total, CH)(f5flat, idx2, w2)

    out = _mlp_call(interp, Wih, bi2, W1h, b12, W2p, b2p, BM, C, H)
    return out.reshape(B, M, 8)[:, :, :3]
